# Initial kernel scaffold; baseline (speedup 1.0000x reference)
import jax
import jax.numpy as jnp
from jax.experimental import pallas as pl


def kernel(x, edge_index, edge_attr, batch, atom_emb1, atom_emb2, edge_emb1, edge_emb2, W1, b1, W2, b2, bn_gamma, bn_beta, proj_W, proj_b):
    raise NotImplementedError("write your pallas kernel here")


# speedup vs baseline: 4.1696x; 4.1696x over previous
import functools

import jax
import jax.numpy as jnp
from jax import lax
from jax.experimental import pallas as pl
from jax.experimental.pallas import tpu as pltpu
from jax.experimental.pallas import tpu_sc as plsc

NUM_LAYER = 5
EMB = 300
NUM_GRAPHS = 256
OUT_DIM = 256

NC = 2
NS = 16
K = 128

NPAD = 10240
NT = 4
TW = 80
SPLITS = (80, 160, 240, 300)
ROWS_PER_TILE = NPAD // NS
ZCH = 64
CW = 32


def _spmm_body(nchunks, h_hbm, src_hbm, dst_hbm, zero_hbm, out_hbm,
               src_v, dst_v, msg_v, zbuf_v, acc, sem):
    c = lax.axis_index("c")
    s = lax.axis_index("s")
    pltpu.sync_copy(dst_hbm.at[pl.ds(s * nchunks, nchunks)], dst_v)
    for phase in range(NT // NC):
        pltpu.sync_copy(zero_hbm, zbuf_v)
        for kk in range(ROWS_PER_TILE // ZCH):
            pltpu.sync_copy(zbuf_v,
                            acc.at[pl.ds(s * ROWS_PER_TILE + kk * ZCH, ZCH)])
        plsc.subcore_barrier()
        t = c + NC * phase
        pltpu.sync_copy(src_hbm.at[t, pl.ds(s * nchunks, nchunks)], src_v)

        def chunk(j, carry):
            pltpu.async_copy(h_hbm.at[src_v.at[j]], msg_v, sem).wait()
            pltpu.sync_copy(msg_v, acc.at[dst_v.at[j]], add=True)
            return carry

        lax.fori_loop(0, nchunks, chunk, 0)
        plsc.subcore_barrier()
        for kk in range(ROWS_PER_TILE // ZCH):
            r = s * ROWS_PER_TILE + kk * ZCH
            pltpu.sync_copy(acc.at[pl.ds(r, ZCH)], zbuf_v)
            pltpu.sync_copy(zbuf_v, out_hbm.at[pl.ds(t * NPAD + r, ZCH)])
        plsc.subcore_barrier()


def _make_spmm(nchunks):
    mesh = plsc.VectorSubcoreMesh(core_axis_name="c", subcore_axis_name="s")
    return pl.kernel(
        functools.partial(_spmm_body, nchunks),
        out_type=jax.ShapeDtypeStruct((NT * NPAD, TW), jnp.float32),
        mesh=mesh,
        compiler_params=pltpu.CompilerParams(use_tc_tiling_on_sc=False),
        scratch_types=[
            pltpu.VMEM((nchunks, K), jnp.int32),
            pltpu.VMEM((nchunks, K), jnp.int32),
            pltpu.VMEM((K, TW), jnp.float32),
            pltpu.VMEM((ZCH, TW), jnp.float32),
            pltpu.VMEM_SHARED((NPAD, TW), jnp.float32),
            pltpu.SemaphoreType.DMA,
        ],
    )


def _counts_body(nchunks, oh_hbm, code_hbm, dst_hbm, zero_hbm, out_hbm,
                 code_v, dst_v, oh_v, zbuf_v, acc, sem):
    c = lax.axis_index("c")
    s = lax.axis_index("s")
    pltpu.sync_copy(zero_hbm, zbuf_v)
    for kk in range(ROWS_PER_TILE // ZCH):
        pltpu.sync_copy(zbuf_v, acc.at[pl.ds(s * ROWS_PER_TILE + kk * ZCH, ZCH)])
    plsc.subcore_barrier()
    row0 = (c * NS + s) * nchunks
    pltpu.sync_copy(code_hbm.at[pl.ds(row0, nchunks)], code_v)
    pltpu.sync_copy(dst_hbm.at[pl.ds(row0, nchunks)], dst_v)

    def chunk(j, carry):
        pltpu.async_copy(oh_hbm.at[code_v.at[j]], oh_v, sem).wait()
        pltpu.sync_copy(oh_v, acc.at[dst_v.at[j]], add=True)
        return carry

    lax.fori_loop(0, nchunks, chunk, 0)
    plsc.subcore_barrier()
    for kk in range(ROWS_PER_TILE // ZCH):
        r = s * ROWS_PER_TILE + kk * ZCH
        pltpu.sync_copy(acc.at[pl.ds(r, ZCH)], zbuf_v)
        pltpu.sync_copy(zbuf_v, out_hbm.at[pl.ds(c * NPAD + r, ZCH)])


def _make_counts(nchunks):
    mesh = plsc.VectorSubcoreMesh(core_axis_name="c", subcore_axis_name="s")
    return pl.kernel(
        functools.partial(_counts_body, nchunks),
        out_type=jax.ShapeDtypeStruct((NC * NPAD, CW), jnp.float32),
        mesh=mesh,
        compiler_params=pltpu.CompilerParams(use_tc_tiling_on_sc=False),
        scratch_types=[
            pltpu.VMEM((nchunks, K), jnp.int32),
            pltpu.VMEM((nchunks, K), jnp.int32),
            pltpu.VMEM((K, CW), jnp.float32),
            pltpu.VMEM((ZCH, CW), jnp.float32),
            pltpu.VMEM_SHARED((NPAD, CW), jnp.float32),
            pltpu.SemaphoreType.DMA,
        ],
    )


BLK = 1024
NBLK = NPAD // BLK


def _join_tables(ref):
    parts = [ref[0], ref[1], ref[2], ref[3][:, :EMB - SPLITS[2]]]
    return jnp.concatenate(parts, axis=1)


def _split_tables(v):
    lo = 0
    outs = []
    for t in range(NT):
        hi = SPLITS[t]
        part = v[:, lo:hi]
        if hi - lo < TW:
            part = jnp.concatenate(
                [part, jnp.zeros((v.shape[0], TW - (hi - lo)), v.dtype)], axis=1)
        outs.append(part)
        lo = hi
    return jnp.stack(outs)


def _init_body(x0_ref, x1_ref, e1_ref, e2_ref, out_ref):
    x0 = x0_ref[...]
    x1 = x1_ref[...]
    oh1 = (lax.broadcasted_iota(jnp.int32, (BLK, e1_ref.shape[0]), 1)
           == x0[:, None]).astype(jnp.float32)
    oh2 = (lax.broadcasted_iota(jnp.int32, (BLK, e2_ref.shape[0]), 1)
           == x1[:, None]).astype(jnp.float32)
    h = (jnp.dot(oh1, e1_ref[...], preferred_element_type=jnp.float32)
         + jnp.dot(oh2, e2_ref[...], preferred_element_type=jnp.float32))
    out_ref[...] = _split_tables(h)


def _mlp_body(last, agg_ref, h_ref, cnt_ref, t_ref, w1_ref, b1_ref, w2_ref,
              b2_ref, g_ref, be_ref, out_ref):
    agg = _join_tables(agg_ref)
    hh = _join_tables(h_ref)
    cnt = cnt_ref[0] + cnt_ref[1]
    t = t_ref[...]
    a = agg + hh + jnp.dot(cnt, t, preferred_element_type=jnp.float32) + t[12:13, :]
    hid = jnp.maximum(jnp.dot(a, w1_ref[...], preferred_element_type=jnp.float32)
                      + b1_ref[...], 0.0)
    o = jnp.dot(hid, w2_ref[...], preferred_element_type=jnp.float32) + b2_ref[...]
    scale = g_ref[...] * (1.0 / jnp.sqrt(1.0 + 1e-5))
    o = o * scale + be_ref[...]
    if not last:
        o = jnp.maximum(o, 0.0)
    out_ref[...] = _split_tables(o)


def _pool_body(h_ref, batch_ref, pw_ref, pb_ref, out_ref, acc_ref):
    i = pl.program_id(0)
    hh = jnp.concatenate([_join_tables(h_ref),
                          jnp.ones((BLK, 4), jnp.float32)], axis=1)
    oh = (lax.broadcasted_iota(jnp.int32, (BLK, NUM_GRAPHS), 1)
          == batch_ref[...][:, None]).astype(jnp.float32)
    part = lax.dot_general(oh, hh, (((0,), (0,)), ((), ())),
                           preferred_element_type=jnp.float32)

    @pl.when(i == 0)
    def _():
        acc_ref[...] = part

    @pl.when(i > 0)
    def _():
        acc_ref[...] += part

    @pl.when(i == NBLK - 1)
    def _():
        acc = acc_ref[...]
        rep = acc[:, :EMB] / jnp.clip(acc[:, EMB:EMB + 1], 1.0, None)
        out_ref[...] = (jnp.dot(rep, pw_ref[...],
                                preferred_element_type=jnp.float32) + pb_ref[...])


def kernel(x, edge_index, edge_attr, batch, atom_emb1, atom_emb2, edge_emb1,
           edge_emb2, W1, b1, W2, b2, bn_gamma, bn_beta, proj_W, proj_b):
    N = x.shape[0]
    E = edge_index.shape[1]

    nchunks = ((-(-E // (NS * K)) + 7) // 8) * 8
    epad = NS * K * nchunks
    cchunks = ((-(-E // (NC * NS * K)) + 7) // 8) * 8
    cpad = NC * NS * K * cchunks

    src = edge_index[0].astype(jnp.int32)
    dst = edge_index[1].astype(jnp.int32)
    code = (edge_attr[:, 0] * 3 + edge_attr[:, 1]).astype(jnp.int32)

    pad_to = max(epad, cpad)
    src_p = jnp.full((pad_to,), N, jnp.int32).at[:E].set(src)
    dst_p = jnp.full((pad_to,), N, jnp.int32).at[:E].set(dst)
    code_p = jnp.full((pad_to,), CW - 1, jnp.int32).at[:E].set(code)

    src_r = src_p[:epad].reshape(NS * nchunks, K)
    src4 = jnp.stack([src_r + t * NPAD for t in range(NT)])
    dst_r = dst_p[:epad].reshape(NS * nchunks, K)
    code_c = code_p[:cpad].reshape(NC * NS * cchunks, K)
    dst_c = dst_p[:cpad].reshape(NC * NS * cchunks, K)

    zero_t = jnp.zeros((ZCH, TW), jnp.float32)
    zero_c = jnp.zeros((ZCH, CW), jnp.float32)
    oh_table = jnp.eye(CW, dtype=jnp.float32)

    x0 = jnp.full((NPAD,), 127, jnp.int32).at[:N].set(x[:, 0].astype(jnp.int32))
    x1 = jnp.full((NPAD,), 127, jnp.int32).at[:N].set(x[:, 1].astype(jnp.int32))
    batch_p = jnp.full((NPAD,), NUM_GRAPHS + 7, jnp.int32).at[:N].set(
        batch.astype(jnp.int32))

    T = (edge_emb1[:, :, None, :] + edge_emb2[:, None, :, :]).reshape(
        NUM_LAYER, 18, EMB)
    T = jnp.concatenate(
        [T, jnp.zeros((NUM_LAYER, CW - 18, EMB), jnp.float32)], axis=1)

    counts2 = _make_counts(cchunks)(oh_table, code_c, dst_c, zero_c)
    cnt3 = counts2.reshape(NC, NPAD, CW)

    h4 = pl.pallas_call(
        _init_body,
        grid=(NBLK,),
        in_specs=[
            pl.BlockSpec((BLK,), lambda i: (i,)),
            pl.BlockSpec((BLK,), lambda i: (i,)),
            pl.BlockSpec(atom_emb1.shape, lambda i: (0, 0)),
            pl.BlockSpec(atom_emb2.shape, lambda i: (0, 0)),
        ],
        out_specs=pl.BlockSpec((NT, BLK, TW), lambda i: (0, i, 0)),
        out_shape=jax.ShapeDtypeStruct((NT, NPAD, TW), jnp.float32),
    )(x0, x1, atom_emb1, atom_emb2)

    spmm = _make_spmm(nchunks)
    b1_2 = b1.reshape(NUM_LAYER, 1, 2 * EMB)
    b2_2 = b2.reshape(NUM_LAYER, 1, EMB)
    g_2 = bn_gamma.reshape(NUM_LAYER, 1, EMB)
    be_2 = bn_beta.reshape(NUM_LAYER, 1, EMB)

    for l in range(NUM_LAYER):
        agg4 = spmm(h4.reshape(NT * NPAD, TW), src4, dst_r, zero_t)
        h4 = pl.pallas_call(
            functools.partial(_mlp_body, l == NUM_LAYER - 1),
            grid=(NBLK,),
            in_specs=[
                pl.BlockSpec((NT, BLK, TW), lambda i: (0, i, 0)),
                pl.BlockSpec((NT, BLK, TW), lambda i: (0, i, 0)),
                pl.BlockSpec((NC, BLK, CW), lambda i: (0, i, 0)),
                pl.BlockSpec((CW, EMB), lambda i: (0, 0)),
                pl.BlockSpec((EMB, 2 * EMB), lambda i: (0, 0)),
                pl.BlockSpec((1, 2 * EMB), lambda i: (0, 0)),
                pl.BlockSpec((2 * EMB, EMB), lambda i: (0, 0)),
                pl.BlockSpec((1, EMB), lambda i: (0, 0)),
                pl.BlockSpec((1, EMB), lambda i: (0, 0)),
                pl.BlockSpec((1, EMB), lambda i: (0, 0)),
            ],
            out_specs=pl.BlockSpec((NT, BLK, TW), lambda i: (0, i, 0)),
            out_shape=jax.ShapeDtypeStruct((NT, NPAD, TW), jnp.float32),
        )(agg4.reshape(NT, NPAD, TW), h4, cnt3, T[l], W1[l], b1_2[l], W2[l],
          b2_2[l], g_2[l], be_2[l])

    out = pl.pallas_call(
        _pool_body,
        grid=(NBLK,),
        in_specs=[
            pl.BlockSpec((NT, BLK, TW), lambda i: (0, i, 0)),
            pl.BlockSpec((BLK,), lambda i: (i,)),
            pl.BlockSpec((EMB, OUT_DIM), lambda i: (0, 0)),
            pl.BlockSpec((1, OUT_DIM), lambda i: (0, 0)),
        ],
        out_specs=pl.BlockSpec((NUM_GRAPHS, OUT_DIM), lambda i: (0, 0)),
        out_shape=jax.ShapeDtypeStruct((NUM_GRAPHS, OUT_DIM), jnp.float32),
        scratch_shapes=[pltpu.VMEM((NUM_GRAPHS, EMB + 4), jnp.float32)],
    )(h4, batch_p, proj_W, proj_b.reshape(1, OUT_DIM))
    return out


# speedup vs baseline: 4.9777x; 1.1938x over previous
import functools

import jax
import jax.numpy as jnp
from jax import lax
from jax.experimental import pallas as pl
from jax.experimental.pallas import tpu as pltpu
from jax.experimental.pallas import tpu_sc as plsc

NUM_LAYER = 5
EMB = 300
NUM_GRAPHS = 256
OUT_DIM = 256

NC = 2
NS = 16
L = 16
K = 128

NPAD = 10240
NT = 4
TW = 80
SPLITS = (80, 160, 240, 300)
ROWS_PER_TILE = NPAD // NS
ZCH = 64
CW = 32
SLAB = 8192


def _zero_acc(s, zero_hbm, zbuf_v, acc):
    pltpu.sync_copy(zero_hbm, zbuf_v)
    for kk in range(ROWS_PER_TILE // ZCH):
        pltpu.sync_copy(zbuf_v, acc.at[pl.ds(s * ROWS_PER_TILE + kk * ZCH, ZCH)])


def _copy_out(s, row0, zbuf_v, acc, out_hbm):
    for kk in range(ROWS_PER_TILE // ZCH):
        r = s * ROWS_PER_TILE + kk * ZCH
        pltpu.sync_copy(acc.at[pl.ds(r, ZCH)], zbuf_v)
        pltpu.sync_copy(zbuf_v, out_hbm.at[pl.ds(row0 + r, ZCH)])


def _spmm_body(nchunks, h_hbm, src_hbm, dst_hbm, zero_hbm, out_hbm,
               src_v, dst_v, msg_v, zbuf_v, acc, sem):
    c = lax.axis_index("c")
    s = lax.axis_index("s")
    pltpu.sync_copy(dst_hbm.at[pl.ds(s * nchunks, nchunks)], dst_v)
    for phase in range(NT // NC):
        _zero_acc(s, zero_hbm, zbuf_v, acc)
        plsc.subcore_barrier()
        t = c + NC * phase
        pltpu.sync_copy(src_hbm.at[t, pl.ds(s * nchunks, nchunks)], src_v)

        def chunk(j, carry):
            pltpu.async_copy(h_hbm.at[src_v.at[j]], msg_v, sem).wait()
            pltpu.sync_copy(msg_v, acc.at[dst_v.at[j]], add=True)
            return carry

        lax.fori_loop(0, nchunks, chunk, 0)
        plsc.subcore_barrier()
        _copy_out(s, t * NPAD, zbuf_v, acc, out_hbm)
        plsc.subcore_barrier()


def _make_spmm(nchunks):
    mesh = plsc.VectorSubcoreMesh(core_axis_name="c", subcore_axis_name="s")
    return pl.kernel(
        functools.partial(_spmm_body, nchunks),
        out_type=jax.ShapeDtypeStruct((NT * NPAD, TW), jnp.float32),
        mesh=mesh,
        compiler_params=pltpu.CompilerParams(use_tc_tiling_on_sc=False),
        scratch_types=[
            pltpu.VMEM((nchunks, K), jnp.int32),
            pltpu.VMEM((nchunks, K), jnp.int32),
            pltpu.VMEM((K, TW), jnp.float32),
            pltpu.VMEM((ZCH, TW), jnp.float32),
            pltpu.VMEM_SHARED((NPAD, TW), jnp.float32),
            pltpu.SemaphoreType.DMA,
        ],
    )


def _counts_body(nslabs, dst_hbm, code_hbm, out_hbm, dslab_v, cslab_v, cnt_v):
    c = lax.axis_index("c")
    s = lax.axis_index("s")
    base = s * ROWS_PER_TILE
    zeros16 = jnp.zeros((L,), jnp.float32)
    ones16 = jnp.ones((L,), jnp.float32)

    def zstep(i, carry):
        cnt_v[pl.ds(i * L, L)] = zeros16
        return carry

    lax.fori_loop(0, ROWS_PER_TILE * CW // L, zstep, 0)

    def slab(sl, carry):
        off = (c * nslabs + sl) * SLAB
        pltpu.sync_copy(dst_hbm.at[pl.ds(off, SLAB)], dslab_v)
        pltpu.sync_copy(code_hbm.at[pl.ds(off, SLAB)], cslab_v)

        def step(i, carry2):
            d = dslab_v[pl.ds(i * L, L)]
            cd = cslab_v[pl.ds(i * L, L)]
            m = (d >= base) & (d < base + ROWS_PER_TILE)
            lidx = jnp.where(m, (d - base) * CW + cd, 0)
            plsc.addupdate_scatter(cnt_v, [lidx], ones16, mask=m)
            return carry2

        lax.fori_loop(0, SLAB // L, step, 0)
        return carry

    lax.fori_loop(0, nslabs, slab, 0)
    pltpu.sync_copy(cnt_v,
                    out_hbm.at[pl.ds((c * NPAD + base) * CW,
                                     ROWS_PER_TILE * CW)])


def _make_counts(nslabs):
    mesh = plsc.VectorSubcoreMesh(core_axis_name="c", subcore_axis_name="s")
    return pl.kernel(
        functools.partial(_counts_body, nslabs),
        out_type=jax.ShapeDtypeStruct((NC * NPAD * CW,), jnp.float32),
        mesh=mesh,
        compiler_params=pltpu.CompilerParams(use_tc_tiling_on_sc=False,
                                             needs_layout_passes=False),
        scratch_types=[
            pltpu.VMEM((SLAB,), jnp.int32),
            pltpu.VMEM((SLAB,), jnp.int32),
            pltpu.VMEM((ROWS_PER_TILE * CW,), jnp.float32),
        ],
    )


BLK = 1024
NBLK = NPAD // BLK


def _join_tables(ref):
    parts = [ref[0], ref[1], ref[2], ref[3][:, :EMB - SPLITS[2]]]
    return jnp.concatenate(parts, axis=1)


def _split_tables(v):
    lo = 0
    outs = []
    for t in range(NT):
        hi = SPLITS[t]
        part = v[:, lo:hi]
        if hi - lo < TW:
            part = jnp.concatenate(
                [part, jnp.zeros((v.shape[0], TW - (hi - lo)), v.dtype)], axis=1)
        outs.append(part)
        lo = hi
    return jnp.stack(outs)


def _init_body(x0_ref, x1_ref, e1_ref, e2_ref, out_ref):
    x0 = x0_ref[...]
    x1 = x1_ref[...]
    oh1 = (lax.broadcasted_iota(jnp.int32, (BLK, e1_ref.shape[0]), 1)
           == x0[:, None]).astype(jnp.float32)
    oh2 = (lax.broadcasted_iota(jnp.int32, (BLK, e2_ref.shape[0]), 1)
           == x1[:, None]).astype(jnp.float32)
    h = (jnp.dot(oh1, e1_ref[...], preferred_element_type=jnp.float32)
         + jnp.dot(oh2, e2_ref[...], preferred_element_type=jnp.float32))
    out_ref[...] = _split_tables(h)


def _mlp_body(last, agg_ref, h_ref, cnt_ref, t_ref, w1_ref, b1_ref, w2_ref,
              b2_ref, g_ref, be_ref, out_ref):
    agg = _join_tables(agg_ref)
    hh = _join_tables(h_ref)
    cnt = cnt_ref[0] + cnt_ref[1]
    t = t_ref[...]
    a = agg + hh + jnp.dot(cnt, t, preferred_element_type=jnp.float32) + t[12:13, :]
    hid = jnp.maximum(jnp.dot(a, w1_ref[...], preferred_element_type=jnp.float32)
                      + b1_ref[...], 0.0)
    o = jnp.dot(hid, w2_ref[...], preferred_element_type=jnp.float32) + b2_ref[...]
    scale = g_ref[...] * (1.0 / jnp.sqrt(1.0 + 1e-5))
    o = o * scale + be_ref[...]
    if not last:
        o = jnp.maximum(o, 0.0)
    out_ref[...] = _split_tables(o)


def _pool_body(h_ref, batch_ref, pw_ref, pb_ref, out_ref, acc_ref):
    i = pl.program_id(0)
    hh = jnp.concatenate([_join_tables(h_ref),
                          jnp.ones((BLK, 4), jnp.float32)], axis=1)
    oh = (lax.broadcasted_iota(jnp.int32, (BLK, NUM_GRAPHS), 1)
          == batch_ref[...][:, None]).astype(jnp.float32)
    part = lax.dot_general(oh, hh, (((0,), (0,)), ((), ())),
                           preferred_element_type=jnp.float32)

    @pl.when(i == 0)
    def _():
        acc_ref[...] = part

    @pl.when(i > 0)
    def _():
        acc_ref[...] += part

    @pl.when(i == NBLK - 1)
    def _():
        acc = acc_ref[...]
        rep = acc[:, :EMB] / jnp.clip(acc[:, EMB:EMB + 1], 1.0, None)
        out_ref[...] = (jnp.dot(rep, pw_ref[...],
                                preferred_element_type=jnp.float32) + pb_ref[...])


def kernel(x, edge_index, edge_attr, batch, atom_emb1, atom_emb2, edge_emb1,
           edge_emb2, W1, b1, W2, b2, bn_gamma, bn_beta, proj_W, proj_b):
    N = x.shape[0]
    E = edge_index.shape[1]

    nchunks = ((-(-E // (NS * K)) + 7) // 8) * 8
    epad = NS * K * nchunks
    nslabs = -(-E // (NC * SLAB))
    cpad = NC * SLAB * nslabs

    src = edge_index[0].astype(jnp.int32)
    dst = edge_index[1].astype(jnp.int32)
    code = (edge_attr[:, 0] * 3 + edge_attr[:, 1]).astype(jnp.int32)

    pad_to = max(epad, cpad)
    src_p = jnp.full((pad_to,), N, jnp.int32).at[:E].set(src)
    dst_p = jnp.full((pad_to,), N, jnp.int32).at[:E].set(dst)
    code_p = jnp.full((pad_to,), CW - 1, jnp.int32).at[:E].set(code)

    src_r = src_p[:epad].reshape(NS * nchunks, K)
    src4 = jnp.stack([src_r + t * NPAD for t in range(NT)])
    dst_r = dst_p[:epad].reshape(NS * nchunks, K)

    zero_t = jnp.zeros((ZCH, TW), jnp.float32)

    x0 = jnp.full((NPAD,), 127, jnp.int32).at[:N].set(x[:, 0].astype(jnp.int32))
    x1 = jnp.full((NPAD,), 127, jnp.int32).at[:N].set(x[:, 1].astype(jnp.int32))
    batch_p = jnp.full((NPAD,), NUM_GRAPHS + 7, jnp.int32).at[:N].set(
        batch.astype(jnp.int32))

    T = (edge_emb1[:, :, None, :] + edge_emb2[:, None, :, :]).reshape(
        NUM_LAYER, 18, EMB)
    T = jnp.concatenate(
        [T, jnp.zeros((NUM_LAYER, CW - 18, EMB), jnp.float32)], axis=1)

    counts2 = _make_counts(nslabs)(dst_p[:cpad], code_p[:cpad])
    cnt3 = counts2.reshape(NC, NPAD, CW)

    h4 = pl.pallas_call(
        _init_body,
        grid=(NBLK,),
        in_specs=[
            pl.BlockSpec((BLK,), lambda i: (i,)),
            pl.BlockSpec((BLK,), lambda i: (i,)),
            pl.BlockSpec(atom_emb1.shape, lambda i: (0, 0)),
            pl.BlockSpec(atom_emb2.shape, lambda i: (0, 0)),
        ],
        out_specs=pl.BlockSpec((NT, BLK, TW), lambda i: (0, i, 0)),
        out_shape=jax.ShapeDtypeStruct((NT, NPAD, TW), jnp.float32),
    )(x0, x1, atom_emb1, atom_emb2)

    spmm = _make_spmm(nchunks)
    b1_2 = b1.reshape(NUM_LAYER, 1, 2 * EMB)
    b2_2 = b2.reshape(NUM_LAYER, 1, EMB)
    g_2 = bn_gamma.reshape(NUM_LAYER, 1, EMB)
    be_2 = bn_beta.reshape(NUM_LAYER, 1, EMB)

    for l in range(NUM_LAYER):
        agg4 = spmm(h4.reshape(NT * NPAD, TW), src4, dst_r, zero_t)
        h4 = pl.pallas_call(
            functools.partial(_mlp_body, l == NUM_LAYER - 1),
            grid=(NBLK,),
            in_specs=[
                pl.BlockSpec((NT, BLK, TW), lambda i: (0, i, 0)),
                pl.BlockSpec((NT, BLK, TW), lambda i: (0, i, 0)),
                pl.BlockSpec((NC, BLK, CW), lambda i: (0, i, 0)),
                pl.BlockSpec((CW, EMB), lambda i: (0, 0)),
                pl.BlockSpec((EMB, 2 * EMB), lambda i: (0, 0)),
                pl.BlockSpec((1, 2 * EMB), lambda i: (0, 0)),
                pl.BlockSpec((2 * EMB, EMB), lambda i: (0, 0)),
                pl.BlockSpec((1, EMB), lambda i: (0, 0)),
                pl.BlockSpec((1, EMB), lambda i: (0, 0)),
                pl.BlockSpec((1, EMB), lambda i: (0, 0)),
            ],
            out_specs=pl.BlockSpec((NT, BLK, TW), lambda i: (0, i, 0)),
            out_shape=jax.ShapeDtypeStruct((NT, NPAD, TW), jnp.float32),
        )(agg4.reshape(NT, NPAD, TW), h4, cnt3, T[l], W1[l], b1_2[l], W2[l],
          b2_2[l], g_2[l], be_2[l])

    out = pl.pallas_call(
        _pool_body,
        grid=(NBLK,),
        in_specs=[
            pl.BlockSpec((NT, BLK, TW), lambda i: (0, i, 0)),
            pl.BlockSpec((BLK,), lambda i: (i,)),
            pl.BlockSpec((EMB, OUT_DIM), lambda i: (0, 0)),
            pl.BlockSpec((1, OUT_DIM), lambda i: (0, 0)),
        ],
        out_specs=pl.BlockSpec((NUM_GRAPHS, OUT_DIM), lambda i: (0, 0)),
        out_shape=jax.ShapeDtypeStruct((NUM_GRAPHS, OUT_DIM), jnp.float32),
        scratch_shapes=[pltpu.VMEM((NUM_GRAPHS, EMB + 4), jnp.float32)],
    )(h4, batch_p, proj_W, proj_b.reshape(1, OUT_DIM))
    return out


# speedup vs baseline: 4.9864x; 1.0018x over previous
import functools

import jax
import jax.numpy as jnp
from jax import lax
from jax.experimental import pallas as pl
from jax.experimental.pallas import tpu as pltpu
from jax.experimental.pallas import tpu_sc as plsc

NUM_LAYER = 5
EMB = 300
NUM_GRAPHS = 256
OUT_DIM = 256

NC = 2
NS = 16
L = 16
K = 128

NPAD = 10240
NT = 4
TW = 80
SPLITS = (80, 160, 240, 300)
ROWS_PER_TILE = NPAD // NS
ZCH = 64
CW = 32
SLAB = 8192


def _spmm_body(nchunks, h_hbm, src_hbm, dst_hbm, zero_hbm, out_hbm,
               src_v, dst_v, msg_v, acc, sem):
    c = lax.axis_index("c")
    s = lax.axis_index("s")
    pltpu.sync_copy(dst_hbm.at[pl.ds(s * nchunks, nchunks)], dst_v)
    for phase in range(NT // NC):
        pltpu.sync_copy(zero_hbm,
                        acc.at[pl.ds(s * ROWS_PER_TILE, ROWS_PER_TILE)])
        plsc.subcore_barrier()
        t = c + NC * phase
        pltpu.sync_copy(src_hbm.at[t, pl.ds(s * nchunks, nchunks)], src_v)

        def chunk(j, carry):
            pltpu.async_copy(h_hbm.at[src_v.at[j]], msg_v, sem).wait()
            pltpu.sync_copy(msg_v, acc.at[dst_v.at[j]], add=True)
            return carry

        lax.fori_loop(0, nchunks, chunk, 0)
        plsc.subcore_barrier()
        r = s * ROWS_PER_TILE
        pltpu.sync_copy(acc.at[pl.ds(r, ROWS_PER_TILE)],
                        out_hbm.at[pl.ds(t * NPAD + r, ROWS_PER_TILE)])
        plsc.subcore_barrier()


def _make_spmm(nchunks):
    mesh = plsc.VectorSubcoreMesh(core_axis_name="c", subcore_axis_name="s")
    return pl.kernel(
        functools.partial(_spmm_body, nchunks),
        out_type=jax.ShapeDtypeStruct((NT * NPAD, TW), jnp.float32),
        mesh=mesh,
        compiler_params=pltpu.CompilerParams(use_tc_tiling_on_sc=False),
        scratch_types=[
            pltpu.VMEM((nchunks, K), jnp.int32),
            pltpu.VMEM((nchunks, K), jnp.int32),
            pltpu.VMEM((K, TW), jnp.float32),
            pltpu.VMEM_SHARED((NPAD, TW), jnp.float32),
            pltpu.SemaphoreType.DMA,
        ],
    )


def _counts_body(nslabs, dst_hbm, code_hbm, out_hbm, dslab_v, cslab_v, cnt_v):
    c = lax.axis_index("c")
    s = lax.axis_index("s")
    base = s * ROWS_PER_TILE
    zeros16 = jnp.zeros((L,), jnp.float32)
    ones16 = jnp.ones((L,), jnp.float32)

    def zstep(i, carry):
        cnt_v[pl.ds(i * L, L)] = zeros16
        return carry

    lax.fori_loop(0, ROWS_PER_TILE * CW // L, zstep, 0)

    def slab(sl, carry):
        off = (c * nslabs + sl) * SLAB
        pltpu.sync_copy(dst_hbm.at[pl.ds(off, SLAB)], dslab_v)
        pltpu.sync_copy(code_hbm.at[pl.ds(off, SLAB)], cslab_v)

        def step(i, carry2):
            d = dslab_v[pl.ds(i * L, L)]
            cd = cslab_v[pl.ds(i * L, L)]
            m = (d >= base) & (d < base + ROWS_PER_TILE)
            lidx = jnp.where(m, (d - base) * CW + cd, 0)
            plsc.addupdate_scatter(cnt_v, [lidx], ones16, mask=m)
            return carry2

        lax.fori_loop(0, SLAB // L, step, 0)
        return carry

    lax.fori_loop(0, nslabs, slab, 0)
    pltpu.sync_copy(cnt_v,
                    out_hbm.at[pl.ds((c * NPAD + base) * CW,
                                     ROWS_PER_TILE * CW)])


def _make_counts(nslabs):
    mesh = plsc.VectorSubcoreMesh(core_axis_name="c", subcore_axis_name="s")
    return pl.kernel(
        functools.partial(_counts_body, nslabs),
        out_type=jax.ShapeDtypeStruct((NC * NPAD * CW,), jnp.float32),
        mesh=mesh,
        compiler_params=pltpu.CompilerParams(use_tc_tiling_on_sc=False,
                                             needs_layout_passes=False),
        scratch_types=[
            pltpu.VMEM((SLAB,), jnp.int32),
            pltpu.VMEM((SLAB,), jnp.int32),
            pltpu.VMEM((ROWS_PER_TILE * CW,), jnp.float32),
        ],
    )


BLK = 1024
NBLK = NPAD // BLK


def _join_tables(ref):
    parts = [ref[0], ref[1], ref[2], ref[3][:, :EMB - SPLITS[2]]]
    return jnp.concatenate(parts, axis=1)


def _split_tables(v):
    lo = 0
    outs = []
    for t in range(NT):
        hi = SPLITS[t]
        part = v[:, lo:hi]
        if hi - lo < TW:
            part = jnp.concatenate(
                [part, jnp.zeros((v.shape[0], TW - (hi - lo)), v.dtype)], axis=1)
        outs.append(part)
        lo = hi
    return jnp.stack(outs)


def _init_body(x0_ref, x1_ref, e1_ref, e2_ref, out_ref):
    x0 = x0_ref[...]
    x1 = x1_ref[...]
    oh1 = (lax.broadcasted_iota(jnp.int32, (BLK, e1_ref.shape[0]), 1)
           == x0[:, None]).astype(jnp.float32)
    oh2 = (lax.broadcasted_iota(jnp.int32, (BLK, e2_ref.shape[0]), 1)
           == x1[:, None]).astype(jnp.float32)
    h = (jnp.dot(oh1, e1_ref[...], preferred_element_type=jnp.float32)
         + jnp.dot(oh2, e2_ref[...], preferred_element_type=jnp.float32))
    out_ref[...] = _split_tables(h)


def _mlp_body(last, agg_ref, h_ref, cnt_ref, t_ref, w1_ref, b1_ref, w2_ref,
              b2_ref, g_ref, be_ref, out_ref):
    agg = _join_tables(agg_ref)
    hh = _join_tables(h_ref)
    cnt = cnt_ref[0] + cnt_ref[1]
    t = t_ref[...]
    a = agg + hh + jnp.dot(cnt, t, preferred_element_type=jnp.float32) + t[12:13, :]
    hid = jnp.maximum(jnp.dot(a, w1_ref[...], preferred_element_type=jnp.float32)
                      + b1_ref[...], 0.0)
    o = jnp.dot(hid, w2_ref[...], preferred_element_type=jnp.float32) + b2_ref[...]
    scale = g_ref[...] * (1.0 / jnp.sqrt(1.0 + 1e-5))
    o = o * scale + be_ref[...]
    if not last:
        o = jnp.maximum(o, 0.0)
    out_ref[...] = _split_tables(o)


def _pool_body(h_ref, batch_ref, pw_ref, pb_ref, out_ref, acc_ref):
    i = pl.program_id(0)
    hh = jnp.concatenate([_join_tables(h_ref),
                          jnp.ones((BLK, 4), jnp.float32)], axis=1)
    oh = (lax.broadcasted_iota(jnp.int32, (BLK, NUM_GRAPHS), 1)
          == batch_ref[...][:, None]).astype(jnp.float32)
    part = lax.dot_general(oh, hh, (((0,), (0,)), ((), ())),
                           preferred_element_type=jnp.float32)

    @pl.when(i == 0)
    def _():
        acc_ref[...] = part

    @pl.when(i > 0)
    def _():
        acc_ref[...] += part

    @pl.when(i == NBLK - 1)
    def _():
        acc = acc_ref[...]
        rep = acc[:, :EMB] / jnp.clip(acc[:, EMB:EMB + 1], 1.0, None)
        out_ref[...] = (jnp.dot(rep, pw_ref[...],
                                preferred_element_type=jnp.float32) + pb_ref[...])


def kernel(x, edge_index, edge_attr, batch, atom_emb1, atom_emb2, edge_emb1,
           edge_emb2, W1, b1, W2, b2, bn_gamma, bn_beta, proj_W, proj_b):
    N = x.shape[0]
    E = edge_index.shape[1]

    nchunks = ((-(-E // (NS * K)) + 7) // 8) * 8
    epad = NS * K * nchunks
    nslabs = -(-E // (NC * SLAB))
    cpad = NC * SLAB * nslabs

    src = edge_index[0].astype(jnp.int32)
    dst = edge_index[1].astype(jnp.int32)
    code = (edge_attr[:, 0] * 3 + edge_attr[:, 1]).astype(jnp.int32)

    pad_to = max(epad, cpad)
    src_p = jnp.full((pad_to,), N, jnp.int32).at[:E].set(src)
    dst_p = jnp.full((pad_to,), N, jnp.int32).at[:E].set(dst)
    code_p = jnp.full((pad_to,), CW - 1, jnp.int32).at[:E].set(code)

    src_r = src_p[:epad].reshape(NS * nchunks, K)
    src4 = jnp.stack([src_r + t * NPAD for t in range(NT)])
    dst_r = dst_p[:epad].reshape(NS * nchunks, K)

    zero_t = jnp.zeros((ROWS_PER_TILE, TW), jnp.float32)

    x0 = jnp.full((NPAD,), 127, jnp.int32).at[:N].set(x[:, 0].astype(jnp.int32))
    x1 = jnp.full((NPAD,), 127, jnp.int32).at[:N].set(x[:, 1].astype(jnp.int32))
    batch_p = jnp.full((NPAD,), NUM_GRAPHS + 7, jnp.int32).at[:N].set(
        batch.astype(jnp.int32))

    T = (edge_emb1[:, :, None, :] + edge_emb2[:, None, :, :]).reshape(
        NUM_LAYER, 18, EMB)
    T = jnp.concatenate(
        [T, jnp.zeros((NUM_LAYER, CW - 18, EMB), jnp.float32)], axis=1)

    counts2 = _make_counts(nslabs)(dst_p[:cpad], code_p[:cpad])
    cnt3 = counts2.reshape(NC, NPAD, CW)

    h4 = pl.pallas_call(
        _init_body,
        grid=(NBLK,),
        in_specs=[
            pl.BlockSpec((BLK,), lambda i: (i,)),
            pl.BlockSpec((BLK,), lambda i: (i,)),
            pl.BlockSpec(atom_emb1.shape, lambda i: (0, 0)),
            pl.BlockSpec(atom_emb2.shape, lambda i: (0, 0)),
        ],
        out_specs=pl.BlockSpec((NT, BLK, TW), lambda i: (0, i, 0)),
        out_shape=jax.ShapeDtypeStruct((NT, NPAD, TW), jnp.float32),
    )(x0, x1, atom_emb1, atom_emb2)

    spmm = _make_spmm(nchunks)
    b1_2 = b1.reshape(NUM_LAYER, 1, 2 * EMB)
    b2_2 = b2.reshape(NUM_LAYER, 1, EMB)
    g_2 = bn_gamma.reshape(NUM_LAYER, 1, EMB)
    be_2 = bn_beta.reshape(NUM_LAYER, 1, EMB)

    for l in range(NUM_LAYER):
        agg4 = spmm(h4.reshape(NT * NPAD, TW), src4, dst_r, zero_t)
        h4 = pl.pallas_call(
            functools.partial(_mlp_body, l == NUM_LAYER - 1),
            grid=(NBLK,),
            in_specs=[
                pl.BlockSpec((NT, BLK, TW), lambda i: (0, i, 0)),
                pl.BlockSpec((NT, BLK, TW), lambda i: (0, i, 0)),
                pl.BlockSpec((NC, BLK, CW), lambda i: (0, i, 0)),
                pl.BlockSpec((CW, EMB), lambda i: (0, 0)),
                pl.BlockSpec((EMB, 2 * EMB), lambda i: (0, 0)),
                pl.BlockSpec((1, 2 * EMB), lambda i: (0, 0)),
                pl.BlockSpec((2 * EMB, EMB), lambda i: (0, 0)),
                pl.BlockSpec((1, EMB), lambda i: (0, 0)),
                pl.BlockSpec((1, EMB), lambda i: (0, 0)),
                pl.BlockSpec((1, EMB), lambda i: (0, 0)),
            ],
            out_specs=pl.BlockSpec((NT, BLK, TW), lambda i: (0, i, 0)),
            out_shape=jax.ShapeDtypeStruct((NT, NPAD, TW), jnp.float32),
        )(agg4.reshape(NT, NPAD, TW), h4, cnt3, T[l], W1[l], b1_2[l], W2[l],
          b2_2[l], g_2[l], be_2[l])

    out = pl.pallas_call(
        _pool_body,
        grid=(NBLK,),
        in_specs=[
            pl.BlockSpec((NT, BLK, TW), lambda i: (0, i, 0)),
            pl.BlockSpec((BLK,), lambda i: (i,)),
            pl.BlockSpec((EMB, OUT_DIM), lambda i: (0, 0)),
            pl.BlockSpec((1, OUT_DIM), lambda i: (0, 0)),
        ],
        out_specs=pl.BlockSpec((NUM_GRAPHS, OUT_DIM), lambda i: (0, 0)),
        out_shape=jax.ShapeDtypeStruct((NUM_GRAPHS, OUT_DIM), jnp.float32),
        scratch_shapes=[pltpu.VMEM((NUM_GRAPHS, EMB + 4), jnp.float32)],
    )(h4, batch_p, proj_W, proj_b.reshape(1, OUT_DIM))
    return out
